# baseline (device time: 125426 ns/iter reference)
import jax
import jax.numpy as jnp
from jax import lax
from jax.experimental import pallas as pl
from jax.experimental.pallas import tpu as pltpu

N_DEV = 4
SQ = 2048
SKV = 2048
D_MODEL = 1024
H_LOC = 8
DH = 128
HD = H_LOC * DH
QB = 128
N_QB = SQ // QB
WIN = 384
GB = 128
GFIX = 32
CHUNK = SQ // N_DEV
COLH = D_MODEL // 2
SCALE = 0.08838834764831843
NEG = -1e9


def _body(q_ref, k_ref, v_ref, wo_ref, out_ref,
          obf, ctx_ref, rs_bufR, rs_bufL, send_sems, recv_sems):
    my_i = lax.axis_index("i")
    left = lax.rem(my_i + N_DEV - 1, N_DEV)
    right = lax.rem(my_i + 1, N_DEV)

    barrier = pltpu.get_barrier_semaphore()
    for nbr in (left, right):
        pl.semaphore_signal(barrier, inc=1, device_id=(nbr,),
                            device_id_type=pl.DeviceIdType.MESH)
    pl.semaphore_wait(barrier, 2)

    r = lax.broadcasted_iota(jnp.int32, (QB, WIN), 0)
    c = lax.broadcasted_iota(jnp.int32, (QB, WIN), 1)
    cg = lax.broadcasted_iota(jnp.int32, (QB, GB), 1)

    def compute_chunk(chunk_id):
        for b in range(CHUNK // QB):
            qb = chunk_id * (CHUNK // QB) + b
            rows = pl.ds(qb * QB, QB)
            lo = jnp.clip(qb * (QB // 128) - 1, 0, (SKV - WIN) // 128) * 128
            mask_win = (jnp.abs(r - c + (qb * QB - lo)) <= 128) | (lo + c < 32)
            mask_glob = (cg < 32) & (qb > 0)

            def h_body(h, carry):
                cols = pl.ds(h * DH, DH)
                qh = q_ref[rows, cols]
                kw = k_ref[pl.ds(lo, WIN), cols]
                sw = lax.dot_general(qh, kw, (((1,), (1,)), ((), ())),
                                     preferred_element_type=jnp.float32)
                sw = jnp.exp(jnp.where(mask_win, sw.astype(jnp.bfloat16),
                                       jnp.bfloat16(NEG)))
                sg = lax.dot_general(qh, k_ref[:GB, cols],
                                     (((1,), (1,)), ((), ())),
                                     preferred_element_type=jnp.float32)
                sg = jnp.exp(jnp.where(mask_glob, sg.astype(jnp.bfloat16),
                                       jnp.bfloat16(NEG)))
                denom = (jnp.sum(sw, axis=1, keepdims=True,
                                 dtype=jnp.float32)
                         + jnp.sum(sg, axis=1, keepdims=True,
                                   dtype=jnp.float32))
                ctx = (jnp.dot(sw, v_ref[pl.ds(lo, WIN), cols],
                               preferred_element_type=jnp.float32)
                       + jnp.dot(sg, v_ref[:GB, cols],
                                 preferred_element_type=jnp.float32)) / denom
                ctx_ref[:, cols] = ctx.astype(jnp.bfloat16)
                return carry

            lax.fori_loop(0, H_LOC, h_body, 0)
            acc = jnp.dot(ctx_ref[:, :], wo_ref[:, :],
                          preferred_element_type=jnp.float32)
            obf[rows, :] = acc.astype(jnp.bfloat16)

        @pl.when(chunk_id == 0)
        def _():
            def hfix_body(h, acc):
                cols = pl.ds(h * DH, DH)
                s = lax.dot_general(q_ref[:GFIX, cols], k_ref[:, cols],
                                    (((1,), (1,)), ((), ())),
                                    preferred_element_type=jnp.float32)
                w = jnp.exp(s)
                ctx = jnp.dot(w.astype(jnp.bfloat16), v_ref[:, cols],
                              preferred_element_type=jnp.float32)
                ctx = ctx / jnp.sum(w, axis=1, keepdims=True)
                return acc + jnp.dot(ctx.astype(jnp.bfloat16),
                                     wo_ref[pl.ds(h * DH, DH), :],
                                     preferred_element_type=jnp.float32)

            accf = lax.fori_loop(0, H_LOC, hfix_body,
                                 jnp.zeros((GFIX, D_MODEL), jnp.float32))
            obf[:GFIX, :] = accf.astype(jnp.bfloat16)

    def rowsd(ch):
        return pl.ds(lax.rem(ch + 2 * N_DEV, N_DEV) * CHUNK, CHUNK)

    def colsd(is_right):
        return pl.ds(0, COLH) if is_right else pl.ds(COLH, COLH)

    def start_rs(s, ch, is_right):
        r = pltpu.make_async_remote_copy(
            src_ref=obf.at[rowsd(ch), colsd(is_right)],
            dst_ref=(rs_bufR if is_right else rs_bufL).at[s],
            send_sem=send_sems.at[s if is_right else 3 + s],
            recv_sem=recv_sems.at[s if is_right else 3 + s],
            device_id=(right if is_right else left,),
            device_id_type=pl.DeviceIdType.MESH,
        )
        r.start()
        return r

    def acc_rs(s, ch, is_right):
        rr, cc = rowsd(ch), colsd(is_right)
        obf[rr, cc] = obf[rr, cc] + (rs_bufR if is_right else rs_bufL)[s]

    compute_chunk(my_i)
    rR = start_rs(0, my_i, True)
    rL = start_rs(0, my_i, False)
    compute_chunk(lax.rem(my_i + 3, N_DEV))
    rR.wait()
    acc_rs(0, my_i + 3, True)
    rR = start_rs(1, my_i + 3, True)
    compute_chunk(lax.rem(my_i + 1, N_DEV))
    rL.wait()
    acc_rs(0, my_i + 1, False)
    rL = start_rs(1, my_i + 1, False)
    compute_chunk(lax.rem(my_i + 2, N_DEV))
    rR.wait()
    acc_rs(1, my_i + 2, True)
    rR = start_rs(2, my_i + 2, True)
    rL.wait()
    acc_rs(1, my_i + 2, False)
    rL = start_rs(2, my_i + 2, False)
    rR.wait()
    acc_rs(2, my_i + 1, True)
    rL.wait()
    acc_rs(2, my_i + 3, False)

    for s in range(N_DEV - 1):
        agR = pltpu.make_async_remote_copy(
            src_ref=obf.at[rowsd(my_i + 1 - s), colsd(True)],
            dst_ref=obf.at[rowsd(my_i + 1 - s), colsd(True)],
            send_sem=send_sems.at[6 + s],
            recv_sem=recv_sems.at[6 + s],
            device_id=(right,),
            device_id_type=pl.DeviceIdType.MESH,
        )
        agL = pltpu.make_async_remote_copy(
            src_ref=obf.at[rowsd(my_i - 1 + s), colsd(False)],
            dst_ref=obf.at[rowsd(my_i - 1 + s), colsd(False)],
            send_sem=send_sems.at[9 + s],
            recv_sem=recv_sems.at[9 + s],
            device_id=(left,),
            device_id_type=pl.DeviceIdType.MESH,
        )
        agR.start()
        agL.start()
        agR.wait()
        agL.wait()

    out_ref[:, :] = obf[:, :].astype(jnp.float32)


def kernel(x, Wq, K_ext, V_ext, Wo):
    my_i = lax.axis_index("i")
    wq = lax.dynamic_slice(Wq, (0, my_i * HD), (Wq.shape[0], HD))
    wo = lax.dynamic_slice(Wo, (my_i * HD, 0), (HD, Wo.shape[1]))

    q2 = (x[0].astype(jnp.bfloat16) @ wq.astype(jnp.bfloat16)) * SCALE
    k2 = K_ext[0].reshape(SKV, HD).astype(jnp.bfloat16)
    v2 = V_ext[0].reshape(SKV, HD).astype(jnp.bfloat16)
    wo2 = wo.astype(jnp.bfloat16)

    out2 = pl.pallas_call(
        _body,
        out_shape=jax.ShapeDtypeStruct((SQ, D_MODEL), jnp.float32),
        in_specs=[pl.BlockSpec(memory_space=pltpu.VMEM)] * 4,
        out_specs=pl.BlockSpec(memory_space=pltpu.VMEM),
        scratch_shapes=[
            pltpu.VMEM((SQ, D_MODEL), jnp.bfloat16),
            pltpu.VMEM((QB, HD), jnp.bfloat16),
            pltpu.VMEM((N_DEV - 1, CHUNK, COLH), jnp.bfloat16),
            pltpu.VMEM((N_DEV - 1, CHUNK, COLH), jnp.bfloat16),
            pltpu.SemaphoreType.DMA((4 * (N_DEV - 1),)),
            pltpu.SemaphoreType.DMA((4 * (N_DEV - 1),)),
        ],
        compiler_params=pltpu.CompilerParams(collective_id=0),
    )(q2, k2, v2, wo2)
    return out2.reshape(1, SQ, D_MODEL)


# device time: 109844 ns/iter; 1.1419x vs baseline; 1.1419x over previous
import jax
import jax.numpy as jnp
from jax import lax
from jax.experimental import pallas as pl
from jax.experimental.pallas import tpu as pltpu

N_DEV = 4
SQ = 2048
SKV = 2048
D_MODEL = 1024
H_LOC = 8
DH = 128
HD = H_LOC * DH
QB = 256
N_QB = SQ // QB
WIN = 512
GB = 128
GFIX = 32
CHUNK = SQ // N_DEV
COLH = D_MODEL // 2
SCALE = 0.08838834764831843
NEG = -1e9


def _body(x_ref, wq_ref, k_ref, v_ref, wo_ref, out_ref,
          obf, qblk_ref, ctx_ref, rs_bufR, rs_bufL, send_sems, recv_sems):
    my_i = lax.axis_index("i")
    left = lax.rem(my_i + N_DEV - 1, N_DEV)
    right = lax.rem(my_i + 1, N_DEV)

    barrier = pltpu.get_barrier_semaphore()
    for nbr in (left, right):
        pl.semaphore_signal(barrier, inc=1, device_id=(nbr,),
                            device_id_type=pl.DeviceIdType.MESH)
    pl.semaphore_wait(barrier, 2)

    r = lax.broadcasted_iota(jnp.int32, (QB, WIN), 0)
    c = lax.broadcasted_iota(jnp.int32, (QB, WIN), 1)
    cg = lax.broadcasted_iota(jnp.int32, (QB, GB), 1)

    def compute_chunk(chunk_id):
        for b in range(CHUNK // QB):
            qb = chunk_id * (CHUNK // QB) + b
            rows = pl.ds(qb * QB, QB)
            lo = jnp.clip(qb * (QB // 128) - 1, 0, (SKV - WIN) // 128) * 128
            mask_win = (jnp.abs(r - c + (qb * QB - lo)) <= 128) | (lo + c < 32)
            mask_glob = (cg < 32) & (qb > 0)

            qv = jnp.dot(x_ref[rows, :], wq_ref[:, :],
                         preferred_element_type=jnp.float32)
            qblk_ref[:, :] = qv.astype(jnp.bfloat16)

            def h_body(h, carry):
                cols = pl.ds(h * DH, DH)
                qh = qblk_ref[:, cols]
                kw = k_ref[pl.ds(lo, WIN), cols]
                sw = lax.dot_general(qh, kw, (((1,), (1,)), ((), ())),
                                     preferred_element_type=jnp.float32)
                sw = jnp.exp(jnp.where(mask_win, sw.astype(jnp.bfloat16),
                                       jnp.bfloat16(NEG)))
                sg = lax.dot_general(qh, k_ref[:GB, cols],
                                     (((1,), (1,)), ((), ())),
                                     preferred_element_type=jnp.float32)
                sg = jnp.exp(jnp.where(mask_glob, sg.astype(jnp.bfloat16),
                                       jnp.bfloat16(NEG)))
                denom = (jnp.sum(sw, axis=1, keepdims=True,
                                 dtype=jnp.float32)
                         + jnp.sum(sg, axis=1, keepdims=True,
                                   dtype=jnp.float32))
                ctx = (jnp.dot(sw, v_ref[pl.ds(lo, WIN), cols],
                               preferred_element_type=jnp.float32)
                       + jnp.dot(sg, v_ref[:GB, cols],
                                 preferred_element_type=jnp.float32)) / denom
                ctx_ref[:, cols] = ctx.astype(jnp.bfloat16)
                return carry

            lax.fori_loop(0, H_LOC, h_body, 0)
            acc = jnp.dot(ctx_ref[:, :], wo_ref[:, :],
                          preferred_element_type=jnp.float32)
            obf[rows, :] = acc.astype(jnp.bfloat16)

        @pl.when(chunk_id == 0)
        def _():
            qf = jnp.dot(x_ref[:GFIX, :], wq_ref[:, :],
                         preferred_element_type=jnp.float32)
            qblk_ref[:GFIX, :] = qf.astype(jnp.bfloat16)

            def hfix_body(h, acc):
                cols = pl.ds(h * DH, DH)
                s = lax.dot_general(qblk_ref[:GFIX, cols], k_ref[:, cols],
                                    (((1,), (1,)), ((), ())),
                                    preferred_element_type=jnp.float32)
                w = jnp.exp(s)
                ctx = jnp.dot(w.astype(jnp.bfloat16), v_ref[:, cols],
                              preferred_element_type=jnp.float32)
                ctx = ctx / jnp.sum(w, axis=1, keepdims=True)
                return acc + jnp.dot(ctx.astype(jnp.bfloat16),
                                     wo_ref[pl.ds(h * DH, DH), :],
                                     preferred_element_type=jnp.float32)

            accf = lax.fori_loop(0, H_LOC, hfix_body,
                                 jnp.zeros((GFIX, D_MODEL), jnp.float32))
            obf[:GFIX, :] = accf.astype(jnp.bfloat16)

    def rowsd(ch):
        return pl.ds(lax.rem(ch + 2 * N_DEV, N_DEV) * CHUNK, CHUNK)

    def colsd(is_right):
        return pl.ds(0, COLH) if is_right else pl.ds(COLH, COLH)

    def start_rs(s, ch, is_right):
        r = pltpu.make_async_remote_copy(
            src_ref=obf.at[rowsd(ch), colsd(is_right)],
            dst_ref=(rs_bufR if is_right else rs_bufL).at[s],
            send_sem=send_sems.at[s if is_right else 3 + s],
            recv_sem=recv_sems.at[s if is_right else 3 + s],
            device_id=(right if is_right else left,),
            device_id_type=pl.DeviceIdType.MESH,
        )
        r.start()
        return r

    def acc_rs(s, ch, is_right):
        rr, cc = rowsd(ch), colsd(is_right)
        obf[rr, cc] = obf[rr, cc] + (rs_bufR if is_right else rs_bufL)[s]

    compute_chunk(my_i)
    rR = start_rs(0, my_i, True)
    rL = start_rs(0, my_i, False)
    compute_chunk(lax.rem(my_i + 3, N_DEV))
    rR.wait()
    acc_rs(0, my_i + 3, True)
    rR = start_rs(1, my_i + 3, True)
    compute_chunk(lax.rem(my_i + 1, N_DEV))
    rL.wait()
    acc_rs(0, my_i + 1, False)
    rL = start_rs(1, my_i + 1, False)
    compute_chunk(lax.rem(my_i + 2, N_DEV))
    rR.wait()
    acc_rs(1, my_i + 2, True)
    rR = start_rs(2, my_i + 2, True)
    rL.wait()
    acc_rs(1, my_i + 2, False)
    rL = start_rs(2, my_i + 2, False)
    rR.wait()
    acc_rs(2, my_i + 1, True)
    rL.wait()
    acc_rs(2, my_i + 3, False)

    for s in range(N_DEV - 1):
        agR = pltpu.make_async_remote_copy(
            src_ref=obf.at[rowsd(my_i + 1 - s), colsd(True)],
            dst_ref=obf.at[rowsd(my_i + 1 - s), colsd(True)],
            send_sem=send_sems.at[6 + s],
            recv_sem=recv_sems.at[6 + s],
            device_id=(right,),
            device_id_type=pl.DeviceIdType.MESH,
        )
        agL = pltpu.make_async_remote_copy(
            src_ref=obf.at[rowsd(my_i - 1 + s), colsd(False)],
            dst_ref=obf.at[rowsd(my_i - 1 + s), colsd(False)],
            send_sem=send_sems.at[9 + s],
            recv_sem=recv_sems.at[9 + s],
            device_id=(left,),
            device_id_type=pl.DeviceIdType.MESH,
        )
        agR.start()
        agL.start()
        agR.wait()
        agL.wait()

    out_ref[:, :] = obf[:, :].astype(jnp.float32)


def kernel(x, Wq, K_ext, V_ext, Wo):
    my_i = lax.axis_index("i")
    wq = lax.dynamic_slice(Wq, (0, my_i * HD), (Wq.shape[0], HD))
    wo = lax.dynamic_slice(Wo, (my_i * HD, 0), (HD, Wo.shape[1]))

    x2 = x[0].astype(jnp.bfloat16)
    wq2 = (wq * SCALE).astype(jnp.bfloat16)
    k2 = K_ext[0].reshape(SKV, HD).astype(jnp.bfloat16)
    v2 = V_ext[0].reshape(SKV, HD).astype(jnp.bfloat16)
    wo2 = wo.astype(jnp.bfloat16)

    out2 = pl.pallas_call(
        _body,
        out_shape=jax.ShapeDtypeStruct((SQ, D_MODEL), jnp.float32),
        in_specs=[pl.BlockSpec(memory_space=pltpu.VMEM)] * 5,
        out_specs=pl.BlockSpec(memory_space=pltpu.VMEM),
        scratch_shapes=[
            pltpu.VMEM((SQ, D_MODEL), jnp.bfloat16),
            pltpu.VMEM((QB, HD), jnp.bfloat16),
            pltpu.VMEM((QB, HD), jnp.bfloat16),
            pltpu.VMEM((N_DEV - 1, CHUNK, COLH), jnp.bfloat16),
            pltpu.VMEM((N_DEV - 1, CHUNK, COLH), jnp.bfloat16),
            pltpu.SemaphoreType.DMA((4 * (N_DEV - 1),)),
            pltpu.SemaphoreType.DMA((4 * (N_DEV - 1),)),
        ],
        compiler_params=pltpu.CompilerParams(collective_id=0),
    )(x2, wq2, k2, v2, wo2)
    return out2.reshape(1, SQ, D_MODEL)


# device time: 106009 ns/iter; 1.1832x vs baseline; 1.0362x over previous
import jax
import jax.numpy as jnp
from jax import lax
from jax.experimental import pallas as pl
from jax.experimental.pallas import tpu as pltpu

N_DEV = 4
SQ = 2048
SKV = 2048
D_MODEL = 1024
H_LOC = 8
DH = 128
HD = H_LOC * DH
QB = 256
N_QB = SQ // QB
WIN = 512
GB = 128
GFIX = 32
CHUNK = SQ // N_DEV
COLH = D_MODEL // 2
SCALE = 0.08838834764831843
NEG = -1e9


def _body(x_ref, wq_ref, k_ref, v_ref, wo_ref, out_ref,
          obf, qblk_ref, ctx_ref, rs_bufR, rs_bufL, send_sems, recv_sems):
    my_i = lax.axis_index("i")
    left = lax.rem(my_i + N_DEV - 1, N_DEV)
    right = lax.rem(my_i + 1, N_DEV)

    barrier = pltpu.get_barrier_semaphore()
    for nbr in (left, right):
        pl.semaphore_signal(barrier, inc=1, device_id=(nbr,),
                            device_id_type=pl.DeviceIdType.MESH)
    pl.semaphore_wait(barrier, 2)

    r = lax.broadcasted_iota(jnp.int32, (QB, WIN), 0)
    c = lax.broadcasted_iota(jnp.int32, (QB, WIN), 1)
    cg = lax.broadcasted_iota(jnp.int32, (QB, GB), 1)

    def compute_chunk(chunk_id):
        for b in range(CHUNK // QB):
            qb = chunk_id * (CHUNK // QB) + b
            rows = pl.ds(qb * QB, QB)
            lo = jnp.clip(qb * (QB // 128) - 1, 0, (SKV - WIN) // 128) * 128
            mask_win = (jnp.abs(r - c + (qb * QB - lo)) <= 128) | (lo + c < 32)
            mask_glob = (cg < 32) & (qb > 0)

            qv = jnp.dot(x_ref[rows, :], wq_ref[:, :],
                         preferred_element_type=jnp.float32)
            qblk_ref[:, :] = qv.astype(jnp.bfloat16)

            def h_body(h, carry):
                cols = pl.ds(h * DH, DH)
                qh = qblk_ref[:, cols]
                kw = k_ref[pl.ds(lo, WIN), cols]
                sw = lax.dot_general(qh, kw, (((1,), (1,)), ((), ())),
                                     preferred_element_type=jnp.float32)
                sw = jnp.exp(jnp.where(mask_win, sw.astype(jnp.bfloat16),
                                       jnp.bfloat16(NEG)))
                sg = lax.dot_general(qh, k_ref[:GB, cols],
                                     (((1,), (1,)), ((), ())),
                                     preferred_element_type=jnp.float32)
                sg = jnp.exp(jnp.where(mask_glob, sg.astype(jnp.bfloat16),
                                       jnp.bfloat16(NEG)))
                denom = (jnp.sum(sw, axis=1, keepdims=True,
                                 dtype=jnp.float32)
                         + jnp.sum(sg, axis=1, keepdims=True,
                                   dtype=jnp.float32))
                ctx = (jnp.dot(sw, v_ref[pl.ds(lo, WIN), cols],
                               preferred_element_type=jnp.float32)
                       + jnp.dot(sg, v_ref[:GB, cols],
                                 preferred_element_type=jnp.float32)) / denom
                ctx_ref[:, cols] = ctx.astype(jnp.bfloat16)
                return carry

            lax.fori_loop(0, H_LOC, h_body, 0)
            acc = jnp.dot(ctx_ref[:, :], wo_ref[:, :],
                          preferred_element_type=jnp.float32)
            obf[rows, :] = acc.astype(jnp.bfloat16)

        @pl.when(chunk_id == 0)
        def _():
            qf = jnp.dot(x_ref[:GFIX, :], wq_ref[:, :],
                         preferred_element_type=jnp.float32)
            qblk_ref[:GFIX, :] = qf.astype(jnp.bfloat16)

            def hfix_body(h, acc):
                cols = pl.ds(h * DH, DH)
                s = lax.dot_general(qblk_ref[:GFIX, cols], k_ref[:, cols],
                                    (((1,), (1,)), ((), ())),
                                    preferred_element_type=jnp.float32)
                w = jnp.exp(s)
                ctx = jnp.dot(w.astype(jnp.bfloat16), v_ref[:, cols],
                              preferred_element_type=jnp.float32)
                ctx = ctx / jnp.sum(w, axis=1, keepdims=True)
                return acc + jnp.dot(ctx.astype(jnp.bfloat16),
                                     wo_ref[pl.ds(h * DH, DH), :],
                                     preferred_element_type=jnp.float32)

            accf = lax.fori_loop(0, H_LOC, hfix_body,
                                 jnp.zeros((GFIX, D_MODEL), jnp.float32))
            obf[:GFIX, :] = accf.astype(jnp.bfloat16)

    def rowsd(ch):
        return pl.ds(lax.rem(ch + 2 * N_DEV, N_DEV) * CHUNK, CHUNK)

    def colsd(is_right):
        return pl.ds(0, COLH) if is_right else pl.ds(COLH, COLH)

    def start_rs(s, ch, is_right):
        r = pltpu.make_async_remote_copy(
            src_ref=obf.at[rowsd(ch), colsd(is_right)],
            dst_ref=(rs_bufR if is_right else rs_bufL).at[s],
            send_sem=send_sems.at[s if is_right else 3 + s],
            recv_sem=recv_sems.at[s if is_right else 3 + s],
            device_id=(right if is_right else left,),
            device_id_type=pl.DeviceIdType.MESH,
        )
        r.start()
        return r

    def acc_rs(s, ch, is_right):
        rr, cc = rowsd(ch), colsd(is_right)
        obf[rr, cc] = obf[rr, cc] + (rs_bufR if is_right else rs_bufL)[s]

    compute_chunk(my_i)
    rR = start_rs(0, my_i, True)
    rL = start_rs(0, my_i, False)
    compute_chunk(lax.rem(my_i + 3, N_DEV))
    rR.wait()
    acc_rs(0, my_i + 3, True)
    rR = start_rs(1, my_i + 3, True)
    compute_chunk(lax.rem(my_i + 1, N_DEV))
    rL.wait()
    acc_rs(0, my_i + 1, False)
    rL = start_rs(1, my_i + 1, False)
    compute_chunk(lax.rem(my_i + 2, N_DEV))
    rR.wait()
    acc_rs(1, my_i + 2, True)
    rR = start_rs(2, my_i + 2, True)
    rL.wait()
    acc_rs(1, my_i + 2, False)
    rL = start_rs(2, my_i + 2, False)
    rR.wait()
    acc_rs(2, my_i + 1, True)
    rL.wait()
    acc_rs(2, my_i + 3, False)

    HALF = CHUNK // 2

    def sub_rows(ch, sub):
        return pl.ds(lax.rem(ch + 2 * N_DEV, N_DEV) * CHUNK + sub * HALF,
                     HALF)

    def ag_start(s, sub, is_right):
        ch = (my_i + 1 - s) if is_right else (my_i - 1 + s)
        rr, cc = sub_rows(ch, sub), colsd(is_right)
        r = pltpu.make_async_remote_copy(
            src_ref=obf.at[rr, cc],
            dst_ref=obf.at[rr, cc],
            send_sem=send_sems.at[(6 if is_right else 12) + 2 * s + sub],
            recv_sem=recv_sems.at[(6 if is_right else 12) + 2 * s + sub],
            device_id=(right if is_right else left,),
            device_id_type=pl.DeviceIdType.MESH,
        )
        r.start()
        return r

    def conv(ch, sub, is_right):
        rr, cc = sub_rows(ch, sub), colsd(is_right)
        out_ref[rr, cc] = obf[rr, cc].astype(jnp.float32)

    ag = {(0, "R"): ag_start(0, 0, True), (1, "R"): ag_start(0, 1, True),
          (0, "L"): ag_start(0, 0, False), (1, "L"): ag_start(0, 1, False)}
    for sub in (0, 1):
        conv(my_i + 1, sub, True)
        conv(my_i + 3, sub, False)
    for s in range(1, N_DEV - 1):
        for sub in (0, 1):
            ag[(sub, "R")].wait()
            ag[(sub, "R")] = ag_start(s, sub, True)
            conv(my_i + 1 - s, sub, True)
            ag[(sub, "L")].wait()
            ag[(sub, "L")] = ag_start(s, sub, False)
            conv(my_i - 1 + s, sub, False)
    for sub in (0, 1):
        ag[(sub, "R")].wait()
        conv(my_i + 2, sub, True)
        ag[(sub, "L")].wait()
        conv(my_i + 2, sub, False)


def kernel(x, Wq, K_ext, V_ext, Wo):
    my_i = lax.axis_index("i")
    wq = lax.dynamic_slice(Wq, (0, my_i * HD), (Wq.shape[0], HD))
    wo = lax.dynamic_slice(Wo, (my_i * HD, 0), (HD, Wo.shape[1]))

    x2 = x[0].astype(jnp.bfloat16)
    wq2 = (wq * SCALE).astype(jnp.bfloat16)
    k2 = K_ext[0].reshape(SKV, HD).astype(jnp.bfloat16)
    v2 = V_ext[0].reshape(SKV, HD).astype(jnp.bfloat16)
    wo2 = wo.astype(jnp.bfloat16)

    out2 = pl.pallas_call(
        _body,
        out_shape=jax.ShapeDtypeStruct((SQ, D_MODEL), jnp.float32),
        in_specs=[pl.BlockSpec(memory_space=pltpu.VMEM)] * 5,
        out_specs=pl.BlockSpec(memory_space=pltpu.VMEM),
        scratch_shapes=[
            pltpu.VMEM((SQ, D_MODEL), jnp.bfloat16),
            pltpu.VMEM((QB, HD), jnp.bfloat16),
            pltpu.VMEM((QB, HD), jnp.bfloat16),
            pltpu.VMEM((N_DEV - 1, CHUNK, COLH), jnp.bfloat16),
            pltpu.VMEM((N_DEV - 1, CHUNK, COLH), jnp.bfloat16),
            pltpu.SemaphoreType.DMA((18,)),
            pltpu.SemaphoreType.DMA((18,)),
        ],
        compiler_params=pltpu.CompilerParams(collective_id=0),
    )(x2, wq2, k2, v2, wo2)
    return out2.reshape(1, SQ, D_MODEL)
